# Initial kernel scaffold; baseline (speedup 1.0000x reference)
import jax
import jax.numpy as jnp
from jax.experimental import pallas as pl


def kernel(x1, x2):
    raise NotImplementedError("write your pallas kernel here")


# speedup vs baseline: 1.1084x; 1.1084x over previous
import jax
import jax.numpy as jnp
from jax import lax
from jax.experimental import pallas as pl
from jax.experimental.pallas import tpu as pltpu


def _kernel(x1_ref, x2_ref, v2_ref):
    bt = x1_ref.shape[0]

    def body(i, carry):
        a = x1_ref[i].astype(jnp.bfloat16)
        b = x2_ref[i].astype(jnp.bfloat16)
        v1 = jnp.dot(a, b, preferred_element_type=jnp.float32)
        v2 = jnp.dot(v1.astype(jnp.bfloat16), a,
                     preferred_element_type=jnp.float32)
        v2_ref[i] = v2
        return carry

    lax.fori_loop(0, bt, body, 0, unroll=bt if bt <= 16 else 8)


def _pick_bt(B, D, itemsize):
    elem_bytes = 2 * 3 * D * D * itemsize
    budget = (48 << 20) - (4 << 20)
    bt = max(1, min(B, budget // elem_bytes))
    if B > 1:
        bt = min(bt, B // 2)
    while B % bt:
        bt -= 1
    return bt


def kernel(x1, x2):
    B, D, D2 = x1.shape
    assert D == D2 and x2.shape == (B, D, D)

    bt = _pick_bt(B, D, jnp.dtype(x1.dtype).itemsize)
    n_steps = B // bt

    itemsize = jnp.dtype(x1.dtype).itemsize
    cost = pl.CostEstimate(
        flops=4 * B * D * D * D,
        transcendentals=0,
        bytes_accessed=3 * B * D * D * itemsize,
    )

    return pl.pallas_call(
        _kernel,
        out_shape=jax.ShapeDtypeStruct((B, D, D), x1.dtype),
        grid=(n_steps,),
        in_specs=[
            pl.BlockSpec((bt, D, D), lambda i: (i, 0, 0)),
            pl.BlockSpec((bt, D, D), lambda i: (i, 0, 0)),
        ],
        out_specs=pl.BlockSpec((bt, D, D), lambda i: (i, 0, 0)),
        compiler_params=pltpu.CompilerParams(
            dimension_semantics=("parallel",),
            vmem_limit_bytes=48 << 20,
        ),
        cost_estimate=cost,
    )(x1, x2)


# speedup vs baseline: 1.1153x; 1.0062x over previous
import jax
import jax.numpy as jnp
from jax import lax
from jax.experimental import pallas as pl
from jax.experimental.pallas import tpu as pltpu


def _kernel(x1_ref, x2_ref, v2_ref):
    bt = x1_ref.shape[0]

    def body(i, carry):
        a = x1_ref[i].astype(jnp.bfloat16)
        b = x2_ref[i].astype(jnp.bfloat16)
        v1 = jnp.dot(a, b, preferred_element_type=jnp.float32)
        v2 = jnp.dot(v1.astype(jnp.bfloat16), a,
                     preferred_element_type=jnp.float32)
        v2_ref[i] = v2
        return carry

    lax.fori_loop(0, bt, body, 0, unroll=bt if bt <= 16 else 8)


def _pick_bt(B, D, itemsize):
    elem_bytes = 2 * 3 * D * D * itemsize
    budget = (56 << 20) - (4 << 20)
    bt = max(1, min(B, budget // elem_bytes))
    if B > 1:
        bt = min(bt, B // 2)
    while B % bt:
        bt -= 1
    return bt


def kernel(x1, x2):
    B, D, D2 = x1.shape
    assert D == D2 and x2.shape == (B, D, D)

    bt = _pick_bt(B, D, jnp.dtype(x1.dtype).itemsize)
    n_steps = B // bt

    itemsize = jnp.dtype(x1.dtype).itemsize
    cost = pl.CostEstimate(
        flops=4 * B * D * D * D,
        transcendentals=0,
        bytes_accessed=3 * B * D * D * itemsize,
    )

    return pl.pallas_call(
        _kernel,
        out_shape=jax.ShapeDtypeStruct((B, D, D), x1.dtype),
        grid=(n_steps,),
        in_specs=[
            pl.BlockSpec((bt, D, D), lambda i: (i, 0, 0)),
            pl.BlockSpec((bt, D, D), lambda i: (i, 0, 0)),
        ],
        out_specs=pl.BlockSpec((bt, D, D), lambda i: (i, 0, 0)),
        compiler_params=pltpu.CompilerParams(
            dimension_semantics=("parallel",),
            vmem_limit_bytes=58 << 20,
        ),
        cost_estimate=cost,
    )(x1, x2)
